# baseline (speedup 1.0000x reference)
import functools

import jax
import jax.numpy as jnp
from jax import lax
from jax.experimental import pallas as pl
from jax.experimental.pallas import tpu as pltpu
from jax.experimental.pallas import tpu_sc as plsc

B, L = 16384, 200
N = B * L
NW = 32
PER_W = N // NW
CHUNK = 25_600
NCHUNK = PER_W // CHUNK
VECS = CHUNK // 16

_mesh = plsc.VectorSubcoreMesh(core_axis_name="c", subcore_axis_name="s")


@functools.partial(
    pl.kernel,
    mesh=_mesh,
    compiler_params=pltpu.CompilerParams(needs_layout_passes=False),
    out_type=jax.ShapeDtypeStruct((N,), jnp.float32),
    scratch_types=[
        pltpu.VMEM((16,), jnp.float32),
        pltpu.VMEM((CHUNK,), jnp.int32),
        pltpu.VMEM((CHUNK,), jnp.float32),
    ],
)
def _lookup(x_hbm, tab_hbm, out_hbm, tab_v, idx_v, out_v):
    wid = lax.axis_index("s") * 2 + lax.axis_index("c")
    base = wid * PER_W
    pltpu.sync_copy(tab_hbm, tab_v)
    tab = tab_v[...]

    def chunk_body(ci, carry):
        off = base + ci * CHUNK
        pltpu.sync_copy(x_hbm.at[pl.ds(off, CHUNK)], idx_v)

        def vec_body(i, c):
            iv = idx_v[pl.ds(i * 16, 16)]
            out_v[pl.ds(i * 16, 16)] = plsc.load_gather(tab_v, [iv])
            return c

        lax.fori_loop(0, VECS, vec_body, 0, unroll=4)
        pltpu.sync_copy(out_v, out_hbm.at[pl.ds(off, CHUNK)])
        return carry

    lax.fori_loop(0, NCHUNK, chunk_body, 0)


def kernel(x, table):
    xf = x.reshape(N).astype(jnp.int32)
    tab16 = jnp.pad(table.reshape(11), (0, 5))
    out = _lookup(xf, tab16)
    return out.reshape(B, L, 1)


# speedup vs baseline: 2.0490x; 2.0490x over previous
import functools

import jax
import jax.numpy as jnp
from jax import lax
from jax.experimental import pallas as pl
from jax.experimental.pallas import tpu as pltpu
from jax.experimental.pallas import tpu_sc as plsc

B, L = 16384, 200
NW = 32
ROWS_W = B // NW
CROWS = 64
NCHUNK = ROWS_W // CROWS
_WINDOWS = tuple(range(0, L - 15, 16)) + (L - 16,)

_mesh = plsc.VectorSubcoreMesh(core_axis_name="c", subcore_axis_name="s")


@functools.partial(
    pl.kernel,
    mesh=_mesh,
    compiler_params=pltpu.CompilerParams(needs_layout_passes=False),
    out_type=jax.ShapeDtypeStruct((B, L), jnp.float32),
    scratch_types=[
        pltpu.VMEM((16,), jnp.float32),
        pltpu.VMEM((CROWS, L), jnp.int32),
        pltpu.VMEM((CROWS, L), jnp.int32),
        pltpu.VMEM((CROWS, L), jnp.float32),
        pltpu.VMEM((CROWS, L), jnp.float32),
        pltpu.SemaphoreType.DMA,
        pltpu.SemaphoreType.DMA,
        pltpu.SemaphoreType.DMA,
        pltpu.SemaphoreType.DMA,
    ],
)
def _lookup(x_hbm, tab_hbm, out_hbm, tab_v, idx0, idx1, out0, out1,
            si0, si1, so0, so1):
    wid = lax.axis_index("s") * 2 + lax.axis_index("c")
    base = wid * ROWS_W
    idx_bufs = (idx0, idx1)
    out_bufs = (out0, out1)
    si = (si0, si1)
    so = (so0, so1)

    pltpu.sync_copy(tab_hbm, tab_v)

    in_handles = [None, None]
    out_handles = [None, None]
    in_handles[0] = pltpu.async_copy(
        x_hbm.at[pl.ds(base, CROWS)], idx_bufs[0], si[0]
    )
    for c in range(NCHUNK):
        b = c & 1
        nb = b ^ 1
        if c + 1 < NCHUNK:
            in_handles[nb] = pltpu.async_copy(
                x_hbm.at[pl.ds(base + (c + 1) * CROWS, CROWS)],
                idx_bufs[nb],
                si[nb],
            )
        in_handles[b].wait()
        if c >= 2:
            out_handles[b].wait()
        ib = idx_bufs[b]
        ob = out_bufs[b]

        def row_body(r, carry, ib=ib, ob=ob):
            for cst in _WINDOWS:
                iv = ib[r, pl.ds(cst, 16)]
                ob[r, pl.ds(cst, 16)] = plsc.load_gather(tab_v, [iv])
            return carry

        lax.fori_loop(0, CROWS, row_body, 0, unroll=2)
        out_handles[b] = pltpu.async_copy(
            ob, out_hbm.at[pl.ds(base + c * CROWS, CROWS)], so[b]
        )
    out_handles[0].wait()
    out_handles[1].wait()


def kernel(x, table):
    tab16 = jnp.pad(table.reshape(11), (0, 5))
    out = _lookup(x, tab16)
    return out[:, :, None]


# speedup vs baseline: 4.4460x; 2.1698x over previous
import functools

import jax
import jax.numpy as jnp
from jax import lax
from jax.experimental import pallas as pl
from jax.experimental.pallas import tpu as pltpu
from jax.experimental.pallas import tpu_sc as plsc

B, L = 16384, 200
N = B * L
NW = 32
QB = 4096
NQ = (B // QB) * L
UNITS_W = NQ // NW
NIN = 3

_mesh = plsc.VectorSubcoreMesh(core_axis_name="c", subcore_axis_name="s")


@functools.partial(
    pl.kernel,
    mesh=_mesh,
    compiler_params=pltpu.CompilerParams(
        needs_layout_passes=False,
        disable_bounds_checks=True,
        disable_semaphore_checks=True,
        skip_device_barrier=True,
    ),
    out_type=jax.ShapeDtypeStruct((N,), jnp.float32),
    scratch_types=[
        pltpu.VMEM((16,), jnp.float32),
        pltpu.VMEM((1, QB), jnp.int32),
        pltpu.VMEM((1, QB), jnp.int32),
        pltpu.VMEM((1, QB), jnp.int32),
        pltpu.VMEM((QB,), jnp.float32),
        pltpu.VMEM((QB,), jnp.float32),
        pltpu.SemaphoreType.DMA,
        pltpu.SemaphoreType.DMA,
        pltpu.SemaphoreType.DMA,
        pltpu.SemaphoreType.DMA,
        pltpu.SemaphoreType.DMA,
    ],
)
def _lookup(xt_hbm, tab_hbm, out_hbm, tab_v, idx0, idx1, idx2, out0, out1,
            si0, si1, si2, so0, so1):
    wid = lax.axis_index("s") * 2 + lax.axis_index("c")
    q0 = wid * UNITS_W
    idx_bufs = (idx0, idx1, idx2)
    out_bufs = (out0, out1)
    si = (si0, si1, si2)
    so = (so0, so1)

    pltpu.sync_copy(tab_hbm, tab_v)

    def unit_coords(u):
        q = q0 + u
        return q // 4, (q % 4) * QB

    in_handles = [None] * NIN
    out_handles = [None, None]

    def start_in(u):
        r = u % NIN
        lq, bq = unit_coords(u)
        in_handles[r] = pltpu.async_copy(
            xt_hbm.at[pl.ds(lq, 1), pl.ds(bq, QB)], idx_bufs[r], si[r]
        )

    for u in range(min(NIN - 1, UNITS_W)):
        start_in(u)
    for u in range(UNITS_W):
        r = u % NIN
        b = u & 1
        if u + NIN - 1 < UNITS_W:
            start_in(u + NIN - 1)
        in_handles[r].wait()
        if u >= 2:
            out_handles[b].wait()
        ib = idx_bufs[r]
        ob = out_bufs[b]

        @plsc.parallel_loop(0, QB, step=16, unroll=8)
        def _win(i, ib=ib, ob=ob):
            iv = ib[0, pl.ds(i, 16)]
            ob[pl.ds(i, 16)] = plsc.load_gather(tab_v, [iv])

        lq, bq = unit_coords(u)
        out_handles[b] = pltpu.async_copy(
            ob, out_hbm.at[pl.ds(lq * B + bq, QB)], so[b]
        )
    out_handles[0].wait()
    out_handles[1].wait()


def kernel(x, table):
    tab16 = jnp.pad(table.reshape(11), (0, 5))
    flat = _lookup(x.T, tab16)
    return flat.reshape(L, B, 1).transpose(1, 0, 2)
